# speedup vs baseline: 1.4144x; 1.4144x over previous
import jax
import jax.numpy as jnp
from jax import lax
from jax.experimental import pallas as pl
from jax.experimental.pallas import tpu as pltpu
from jax.experimental.pallas import tpu_sc as plsc

_VOCAB = 100000
_EMB = 1024
_SCALE = 32.0
_NC = 2
_NS = 16
_NW = _NC * _NS
_B_TOT = 4 * 4096
_B_PER_W = _B_TOT // _NW
_CHUNK = 32
_NCHUNK = _B_PER_W // _CHUNK
_LANES = 16
_VPR = _EMB // _LANES


def _emb_body(tokens_hbm, table_hbm, out_hbm, idx_v, rows_v, gsem, wsem):
    wid = lax.axis_index("s") * _NC + lax.axis_index("c")
    base = wid * _B_PER_W
    pltpu.sync_copy(tokens_hbm.at[pl.ds(base, _B_PER_W)], idx_v)

    def start_gather(g):
        b = g % 2
        return pltpu.async_copy(
            table_hbm.at[idx_v.at[pl.ds(g * _CHUNK, _CHUNK)]],
            rows_v.at[b],
            gsem,
        )

    gh = {0: start_gather(0), 1: start_gather(1)}
    wh = {}
    for g in range(_NCHUNK):
        b = g % 2
        gh[g].wait()

        def row_body(r, carry, _b=b):
            for c in range(_VPR):
                sl = (_b, r, pl.ds(c * _LANES, _LANES))
                rows_v[sl] = rows_v[sl] * _SCALE
            return carry

        lax.fori_loop(0, _CHUNK, row_body, 0)
        wh[g] = pltpu.async_copy(
            rows_v.at[b],
            out_hbm.at[pl.ds(base + g * _CHUNK, _CHUNK)],
            wsem,
        )
        if g + 2 < _NCHUNK:
            wh[g].wait()
            gh[g + 2] = start_gather(g + 2)
    wh[_NCHUNK - 2].wait()
    wh[_NCHUNK - 1].wait()


_emb_kernel = pl.kernel(
    _emb_body,
    out_type=jax.ShapeDtypeStruct((_B_TOT, _EMB), jnp.float32),
    mesh=plsc.VectorSubcoreMesh(
        core_axis_name="c", subcore_axis_name="s",
        num_cores=_NC, num_subcores=_NS,
    ),
    scratch_types=[
        pltpu.VMEM((_B_PER_W,), jnp.int32),
        pltpu.VMEM((2, _CHUNK, _EMB), jnp.float32),
        pltpu.SemaphoreType.DMA,
        pltpu.SemaphoreType.DMA,
    ],
)


def kernel(tokens, table):
    b, s = tokens.shape
    flat = jnp.reshape(tokens.astype(jnp.int32), (b * s,))
    out = _emb_kernel(flat, table)
    return jnp.reshape(out, (b, s, _EMB))


# speedup vs baseline: 1.5040x; 1.0634x over previous
import jax
import jax.numpy as jnp
from jax import lax
from jax.experimental import pallas as pl
from jax.experimental.pallas import tpu as pltpu
from jax.experimental.pallas import tpu_sc as plsc

_VOCAB = 100000
_EMB = 1024
_SCALE = 32.0
_NC = 2
_NS = 16
_NW = _NC * _NS
_B_TOT = 4 * 4096
_B_PER_W = _B_TOT // _NW
_CHUNK = 32
_NCHUNK = _B_PER_W // _CHUNK
_NBUF = 3
_LANES = 16
_VPR = _EMB // _LANES


def _emb_body(tokens_hbm, table_hbm, out_hbm, idx_v, rows_v, gsem, wsem):
    wid = lax.axis_index("s") * _NC + lax.axis_index("c")
    base = wid * _B_PER_W
    pltpu.sync_copy(tokens_hbm.at[pl.ds(base, _B_PER_W)], idx_v)

    def start_gather(g):
        b = g % _NBUF
        return pltpu.async_copy(
            table_hbm.at[idx_v.at[pl.ds(g * _CHUNK, _CHUNK)]],
            rows_v.at[b],
            gsem,
        )

    gh = {g: start_gather(g) for g in range(_NBUF)}
    wh = {}
    for g in range(_NCHUNK):
        b = g % _NBUF
        if g >= 2:
            wh[g - 2].wait()
            if g + 1 < _NCHUNK:
                gh[g + 1] = start_gather(g + 1)
        gh[g].wait()

        def row_body(r, carry, _b=b):
            for c in range(_VPR):
                sl = (_b, r, pl.ds(c * _LANES, _LANES))
                rows_v[sl] = rows_v[sl] * _SCALE
            return carry

        lax.fori_loop(0, _CHUNK, row_body, 0)
        wh[g] = pltpu.async_copy(
            rows_v.at[b],
            out_hbm.at[pl.ds(base + g * _CHUNK, _CHUNK)],
            wsem,
        )
    wh[_NCHUNK - 2].wait()
    wh[_NCHUNK - 1].wait()


_emb_kernel = pl.kernel(
    _emb_body,
    out_type=jax.ShapeDtypeStruct((_B_TOT, _EMB), jnp.float32),
    mesh=plsc.VectorSubcoreMesh(
        core_axis_name="c", subcore_axis_name="s",
        num_cores=_NC, num_subcores=_NS,
    ),
    scratch_types=[
        pltpu.VMEM((_B_PER_W,), jnp.int32),
        pltpu.VMEM((_NBUF, _CHUNK, _EMB), jnp.float32),
        pltpu.SemaphoreType.DMA,
        pltpu.SemaphoreType.DMA,
    ],
)


def kernel(tokens, table):
    b, s = tokens.shape
    flat = jnp.reshape(tokens.astype(jnp.int32), (b * s,))
    out = _emb_kernel(flat, table)
    return jnp.reshape(out, (b, s, _EMB))


# speedup vs baseline: 1.5083x; 1.0029x over previous
import jax
import jax.numpy as jnp
from jax import lax
from jax.experimental import pallas as pl
from jax.experimental.pallas import tpu as pltpu
from jax.experimental.pallas import tpu_sc as plsc

_VOCAB = 100000
_EMB = 1024
_SCALE = 32.0
_NC = 2
_NS = 16
_NW = _NC * _NS
_B_TOT = 4 * 4096
_B_PER_W = _B_TOT // _NW
_CHUNK = 32
_NCHUNK = _B_PER_W // _CHUNK
_NBUF = 3
_LANES = 16
_VPR = _EMB // _LANES


def _emb_body(tokens_hbm, table_hbm, out_hbm, idx_v, rows_v, gsem, wsem):
    wid = lax.axis_index("s") * _NC + lax.axis_index("c")
    base = wid * _B_PER_W
    pltpu.sync_copy(tokens_hbm.at[pl.ds(base, _CHUNK)], idx_v.at[pl.ds(0, _CHUNK)])

    def start_gather(g):
        b = g % _NBUF
        return pltpu.async_copy(
            table_hbm.at[idx_v.at[pl.ds(g * _CHUNK, _CHUNK)]],
            rows_v.at[b],
            gsem,
        )

    gh = {0: start_gather(0)}
    pltpu.sync_copy(
        tokens_hbm.at[pl.ds(base + _CHUNK, _B_PER_W - _CHUNK)],
        idx_v.at[pl.ds(_CHUNK, _B_PER_W - _CHUNK)],
    )
    for g in range(1, _NBUF):
        gh[g] = start_gather(g)
    wh = {}
    for g in range(_NCHUNK):
        b = g % _NBUF
        if g >= 2:
            wh[g - 2].wait()
            if g + 1 < _NCHUNK:
                gh[g + 1] = start_gather(g + 1)
        gh[g].wait()

        def row_body(r, carry, _b=b):
            for c in range(_VPR):
                sl = (_b, r, pl.ds(c * _LANES, _LANES))
                rows_v[sl] = rows_v[sl] * _SCALE
            return carry

        lax.fori_loop(0, _CHUNK, row_body, 0)
        wh[g] = pltpu.async_copy(
            rows_v.at[b],
            out_hbm.at[pl.ds(base + g * _CHUNK, _CHUNK)],
            wsem,
        )
    wh[_NCHUNK - 2].wait()
    wh[_NCHUNK - 1].wait()


_emb_kernel = pl.kernel(
    _emb_body,
    out_type=jax.ShapeDtypeStruct((_B_TOT, _EMB), jnp.float32),
    mesh=plsc.VectorSubcoreMesh(
        core_axis_name="c", subcore_axis_name="s",
        num_cores=_NC, num_subcores=_NS,
    ),
    scratch_types=[
        pltpu.VMEM((_B_PER_W,), jnp.int32),
        pltpu.VMEM((_NBUF, _CHUNK, _EMB), jnp.float32),
        pltpu.SemaphoreType.DMA,
        pltpu.SemaphoreType.DMA,
    ],
)


def kernel(tokens, table):
    b, s = tokens.shape
    flat = jnp.reshape(tokens.astype(jnp.int32), (b * s,))
    out = _emb_kernel(flat, table)
    return jnp.reshape(out, (b, s, _EMB))
